# baseline (speedup 1.0000x reference)
import functools

import jax
import jax.numpy as jnp
import numpy as np
from jax import lax
from jax.experimental import pallas as pl
from jax.experimental.pallas import tpu as pltpu
from jax.experimental.pallas import tpu_sc as plsc

N_REF = 8192
N_QUERY = 2048
RADIUS2 = 0.25

BM = 256
NW = 32
ROWS_PER_W = N_QUERY // NW
WORDS_PER_ROW = N_REF // 32
GROUPS = WORDS_PER_ROW // 16


def _mask_kernel(q_ref, r_ref, rsqt_ref, mask_ref, packed_ref, cnt_ref):
    q = q_ref[...]
    r = r_ref[...]
    dot = lax.dot_general(q, r, (((1,), (1,)), ((), ())),
                          precision=lax.Precision.DEFAULT)
    q_sq = jnp.sum(q * q, axis=1, keepdims=True)
    dist2 = q_sq + rsqt_ref[...] - 2.0 * dot
    mask = dist2 <= RADIUS2
    mask_ref[...] = mask
    mf = jnp.where(mask, 1.0, 0.0)
    ones_col = jnp.ones((N_REF, 1), jnp.float32)
    counts = lax.dot_general(mf, ones_col, (((1,), (0,)), ((), ())),
                             precision=lax.Precision.DEFAULT)
    cnt_ref[...] = jnp.broadcast_to(counts.astype(jnp.int32), (BM, 16))
    acc = jnp.where(mask[:, 0:WORDS_PER_ROW], jnp.int32(1), jnp.int32(0))
    for s in range(1, 32):
        bit = jnp.int32(np.uint32(1 << s).view(np.int32))
        acc = acc + jnp.where(
            mask[:, s * WORDS_PER_ROW:(s + 1) * WORDS_PER_ROW],
            bit, jnp.int32(0))
    packed_ref[...] = acc


def _offsets_kernel(cnt_ref, out_ref):
    cnt = cnt_ref[...].astype(jnp.float32)
    k = lax.broadcasted_iota(jnp.int32, (128, 128), 0)
    l = lax.broadcasted_iota(jnp.int32, (128, 128), 1)
    tri = (k <= l).astype(jnp.float32)
    incl = jnp.dot(cnt, tri, precision=lax.Precision.HIGHEST)
    i = lax.broadcasted_iota(jnp.int32, (16, 16), 0)
    j = lax.broadcasted_iota(jnp.int32, (16, 16), 1)
    strict = (j < i).astype(jnp.float32)
    row_tot = incl[:, 127:128]
    row_off = jnp.dot(strict, row_tot, precision=lax.Precision.HIGHEST)
    excl = row_off + incl - cnt
    out_ref[0:16, :] = excl.astype(jnp.int32)
    total = row_off[15:16, 0:1] + incl[15:16, 127:128]
    out_ref[16:17, :] = jnp.broadcast_to(total, (1, 128)).astype(jnp.int32)


def _sc_body(packed_hbm, cnt_hbm, out_hbm, mbits, cvm,
             rowbuf_a, rowbuf_b, ssem_a, ssem_b):
    info = plsc.get_sparse_core_info()
    nc = info.num_cores
    wid = lax.axis_index("s") * nc + lax.axis_index("c")
    base_row = wid * ROWS_PER_W
    lanes = lax.iota(jnp.int32, 16)
    ones16 = jnp.ones((16,), jnp.int32)

    pltpu.sync_copy(
        packed_hbm.at[pl.ds(base_row * WORDS_PER_ROW,
                            ROWS_PER_W * WORDS_PER_ROW)], mbits)
    pltpu.sync_copy(
        cnt_hbm.at[pl.ds(base_row * 16, ROWS_PER_W * 16)], cvm)

    def step(r, row, rowbuf, ssem):
        @pl.when(r >= 2)
        def _():
            pltpu.make_async_copy(rowbuf, out_hbm.at[row - 2], ssem).wait()

        cntv = cvm[pl.ds(r * 16, 16)]
        wbase = r * WORDS_PER_ROW

        @plsc.parallel_loop(0, N_REF // 16, 1, unroll=8,
                            carry=(lanes, -ones16, cntv + lanes))
        def body(c, carry_in):
            jv, trunm1, cj = carry_in
            g = c & (GROUPS - 1)
            s = c >> 4
            w = mbits[pl.ds(wbase + g * 16, 16)]
            m01 = (w >> s) & 1
            mb = m01 > 0
            incl = plsc.cumsum(m01)
            pos = jnp.where(mb, trunm1 + incl, cj - incl)
            plsc.store_scatter(rowbuf, [pos], jv)
            pc = plsc.all_reduce_population_count(mb)
            return jv + 16, trunm1 + pc, cj + (16 - pc)

        pltpu.async_copy(rowbuf, out_hbm.at[row], ssem)

    def do_row(r, carry):
        row = base_row + r

        @pl.when((r & 1) == 0)
        def _():
            step(r, row, rowbuf_a, ssem_a)

        @pl.when((r & 1) == 1)
        def _():
            step(r, row, rowbuf_b, ssem_b)

        return carry

    lax.fori_loop(0, ROWS_PER_W, do_row, 0)

    pltpu.make_async_copy(
        rowbuf_a, out_hbm.at[base_row + ROWS_PER_W - 2], ssem_a).wait()
    pltpu.make_async_copy(
        rowbuf_b, out_hbm.at[base_row + ROWS_PER_W - 1], ssem_b).wait()


@functools.partial(jax.jit, static_argnums=())
def kernel(ref_positions, query_positions):
    rsqt = jnp.sum(ref_positions * ref_positions, axis=1, keepdims=True).T

    grid = N_QUERY // BM
    mask, packed, counts_spl = pl.pallas_call(
        _mask_kernel,
        grid=(grid,),
        in_specs=[
            pl.BlockSpec((BM, 3), lambda i: (i, 0)),
            pl.BlockSpec((N_REF, 3), lambda i: (0, 0)),
            pl.BlockSpec((1, N_REF), lambda i: (0, 0)),
        ],
        out_specs=[
            pl.BlockSpec((BM, N_REF), lambda i: (i, 0)),
            pl.BlockSpec((BM, WORDS_PER_ROW), lambda i: (i, 0)),
            pl.BlockSpec((BM, 16), lambda i: (i, 0)),
        ],
        out_shape=[
            jax.ShapeDtypeStruct((N_QUERY, N_REF), jnp.bool_),
            jax.ShapeDtypeStruct((N_QUERY, WORDS_PER_ROW), jnp.int32),
            jax.ShapeDtypeStruct((N_QUERY, 16), jnp.int32),
        ],
    )(query_positions, ref_positions, rsqt)

    counts = counts_spl[:, 0]

    offsets_2d = pl.pallas_call(
        _offsets_kernel,
        out_shape=jax.ShapeDtypeStruct((17, 128), jnp.int32),
    )(counts.reshape(16, 128))
    offsets = offsets_2d.reshape(-1)[: N_QUERY + 1]

    mesh = plsc.VectorSubcoreMesh(core_axis_name="c", subcore_axis_name="s")
    neighbor_index = pl.kernel(
        _sc_body,
        out_type=jax.ShapeDtypeStruct((N_QUERY, N_REF), jnp.int32),
        mesh=mesh,
        compiler_params=pltpu.CompilerParams(needs_layout_passes=False),
        scratch_types=[
            pltpu.VMEM((ROWS_PER_W * WORDS_PER_ROW,), jnp.int32),
            pltpu.VMEM((ROWS_PER_W * 16,), jnp.int32),
            pltpu.VMEM((N_REF,), jnp.int32),
            pltpu.VMEM((N_REF,), jnp.int32),
            pltpu.SemaphoreType.DMA,
            pltpu.SemaphoreType.DMA,
        ],
    )(packed.reshape(-1), counts_spl.reshape(-1))

    return neighbor_index, counts, offsets, mask


# speedup vs baseline: 1.0469x; 1.0469x over previous
import functools

import jax
import jax.numpy as jnp
import numpy as np
from jax import lax
from jax.experimental import pallas as pl
from jax.experimental.pallas import tpu as pltpu
from jax.experimental.pallas import tpu_sc as plsc

N_REF = 8192
N_QUERY = 2048
RADIUS2 = 0.25

BM = 256
NW = 32
ROWS_PER_W = N_QUERY // NW
WORDS_PER_ROW = N_REF // 32
GROUPS = WORDS_PER_ROW // 16


def _mask_kernel(q_ref, r_ref, rsqt_ref, mask_ref, packed_ref, cnt_ref):
    q = q_ref[...]
    r = r_ref[...]
    dot = lax.dot_general(q, r, (((1,), (1,)), ((), ())),
                          precision=lax.Precision.DEFAULT)
    q_sq = jnp.sum(q * q, axis=1, keepdims=True)
    dist2 = q_sq + rsqt_ref[...] - 2.0 * dot
    mask = dist2 <= RADIUS2
    mask_ref[...] = mask
    mf = jnp.where(mask, 1.0, 0.0)
    ones_col = jnp.ones((N_REF, 1), jnp.float32)
    counts = lax.dot_general(mf, ones_col, (((1,), (0,)), ((), ())),
                             precision=lax.Precision.DEFAULT)
    cnt_ref[...] = jnp.broadcast_to(counts.astype(jnp.int32), (BM, 16))
    acc = jnp.where(mask[:, 0:WORDS_PER_ROW], jnp.int32(1), jnp.int32(0))
    for s in range(1, 32):
        bit = jnp.int32(np.uint32(1 << s).view(np.int32))
        acc = acc + jnp.where(
            mask[:, s * WORDS_PER_ROW:(s + 1) * WORDS_PER_ROW],
            bit, jnp.int32(0))
    packed_ref[...] = acc


def _offsets_kernel(cnt_ref, out_ref):
    cnt = cnt_ref[...].astype(jnp.float32)
    k = lax.broadcasted_iota(jnp.int32, (128, 128), 0)
    l = lax.broadcasted_iota(jnp.int32, (128, 128), 1)
    tri = (k <= l).astype(jnp.float32)
    incl = jnp.dot(cnt, tri, precision=lax.Precision.HIGHEST)
    i = lax.broadcasted_iota(jnp.int32, (16, 16), 0)
    j = lax.broadcasted_iota(jnp.int32, (16, 16), 1)
    strict = (j < i).astype(jnp.float32)
    row_tot = incl[:, 127:128]
    row_off = jnp.dot(strict, row_tot, precision=lax.Precision.HIGHEST)
    excl = row_off + incl - cnt
    out_ref[0:16, :] = excl.astype(jnp.int32)
    total = row_off[15:16, 0:1] + incl[15:16, 127:128]
    out_ref[16:17, :] = jnp.broadcast_to(total, (1, 128)).astype(jnp.int32)


def _sc_body(packed_hbm, cnt_hbm, out_hbm, mbits, cvm,
             rowbuf_a, rowbuf_b, ssem_a, ssem_b):
    info = plsc.get_sparse_core_info()
    nc = info.num_cores
    wid = lax.axis_index("s") * nc + lax.axis_index("c")
    base_row = wid * ROWS_PER_W
    lanes = lax.iota(jnp.int32, 16)
    ones16 = jnp.ones((16,), jnp.int32)

    pltpu.sync_copy(packed_hbm.at[pl.ds(base_row, ROWS_PER_W), :], mbits)
    pltpu.sync_copy(cnt_hbm.at[pl.ds(base_row, ROWS_PER_W), :], cvm)

    def step(r, row, rowbuf, ssem):
        @pl.when(r >= 2)
        def _():
            pltpu.make_async_copy(rowbuf, out_hbm.at[row - 2], ssem).wait()

        cntv = cvm[r, pl.ds(0, 16)]

        @plsc.parallel_loop(0, N_REF // 16, 1, unroll=8,
                            carry=(lanes, -ones16, cntv + lanes))
        def body(c, carry_in):
            jv, trunm1, cj = carry_in
            g = c & (GROUPS - 1)
            s = c >> 4
            w = mbits[r, pl.ds(g * 16, 16)]
            m01 = (w >> s) & 1
            mb = m01 > 0
            incl = plsc.cumsum(m01)
            pos = jnp.where(mb, trunm1 + incl, cj - incl)
            plsc.store_scatter(rowbuf, [pos], jv)
            pc = plsc.all_reduce_population_count(mb)
            return jv + 16, trunm1 + pc, cj + (16 - pc)

        pltpu.async_copy(rowbuf, out_hbm.at[row], ssem)

    def do_row(r, carry):
        row = base_row + r

        @pl.when((r & 1) == 0)
        def _():
            step(r, row, rowbuf_a, ssem_a)

        @pl.when((r & 1) == 1)
        def _():
            step(r, row, rowbuf_b, ssem_b)

        return carry

    lax.fori_loop(0, ROWS_PER_W, do_row, 0)

    pltpu.make_async_copy(
        rowbuf_a, out_hbm.at[base_row + ROWS_PER_W - 2], ssem_a).wait()
    pltpu.make_async_copy(
        rowbuf_b, out_hbm.at[base_row + ROWS_PER_W - 1], ssem_b).wait()


@functools.partial(jax.jit, static_argnums=())
def kernel(ref_positions, query_positions):
    rsqt = jnp.sum(ref_positions * ref_positions, axis=1, keepdims=True).T

    grid = N_QUERY // BM
    mask, packed, counts_spl = pl.pallas_call(
        _mask_kernel,
        grid=(grid,),
        in_specs=[
            pl.BlockSpec((BM, 3), lambda i: (i, 0)),
            pl.BlockSpec((N_REF, 3), lambda i: (0, 0)),
            pl.BlockSpec((1, N_REF), lambda i: (0, 0)),
        ],
        out_specs=[
            pl.BlockSpec((BM, N_REF), lambda i: (i, 0)),
            pl.BlockSpec((BM, WORDS_PER_ROW), lambda i: (i, 0)),
            pl.BlockSpec((BM, 16), lambda i: (i, 0)),
        ],
        out_shape=[
            jax.ShapeDtypeStruct((N_QUERY, N_REF), jnp.bool_),
            jax.ShapeDtypeStruct((N_QUERY, WORDS_PER_ROW), jnp.int32),
            jax.ShapeDtypeStruct((N_QUERY, 16), jnp.int32),
        ],
    )(query_positions, ref_positions, rsqt)

    counts = counts_spl[:, 0]

    offsets_2d = pl.pallas_call(
        _offsets_kernel,
        out_shape=jax.ShapeDtypeStruct((17, 128), jnp.int32),
    )(counts.reshape(16, 128))
    offsets = offsets_2d.reshape(-1)[: N_QUERY + 1]

    mesh = plsc.VectorSubcoreMesh(core_axis_name="c", subcore_axis_name="s")
    neighbor_index = pl.kernel(
        _sc_body,
        out_type=jax.ShapeDtypeStruct((N_QUERY, N_REF), jnp.int32),
        mesh=mesh,
        compiler_params=pltpu.CompilerParams(needs_layout_passes=False),
        scratch_types=[
            pltpu.VMEM((ROWS_PER_W, WORDS_PER_ROW), jnp.int32),
            pltpu.VMEM((ROWS_PER_W, 16), jnp.int32),
            pltpu.VMEM((N_REF,), jnp.int32),
            pltpu.VMEM((N_REF,), jnp.int32),
            pltpu.SemaphoreType.DMA,
            pltpu.SemaphoreType.DMA,
        ],
    )(packed, counts_spl)

    return neighbor_index, counts, offsets, mask


# speedup vs baseline: 1.0539x; 1.0067x over previous
import functools

import jax
import jax.numpy as jnp
import numpy as np
from jax import lax
from jax.experimental import pallas as pl
from jax.experimental.pallas import tpu as pltpu
from jax.experimental.pallas import tpu_sc as plsc

N_REF = 8192
N_QUERY = 2048
RADIUS2 = 0.25

BM = 256
NW = 32
ROWS_PER_W = N_QUERY // NW
WORDS_PER_ROW = N_REF // 32
GROUPS = WORDS_PER_ROW // 16


def _mask_kernel(q_ref, r_ref, rsqt_ref, mask_ref, packed_ref, cnt_ref):
    q = q_ref[...]
    r = r_ref[...]
    dot = lax.dot_general(q, r, (((1,), (1,)), ((), ())),
                          precision=lax.Precision.DEFAULT)
    q_sq = jnp.sum(q * q, axis=1, keepdims=True)
    dist2 = q_sq + rsqt_ref[...] - 2.0 * dot
    mask = dist2 <= RADIUS2
    mask_ref[...] = mask
    mf = jnp.where(mask, 1.0, 0.0)
    ones_col = jnp.ones((N_REF, 1), jnp.float32)
    counts = lax.dot_general(mf, ones_col, (((1,), (0,)), ((), ())),
                             precision=lax.Precision.DEFAULT)
    cnt_ref[...] = jnp.broadcast_to(counts.astype(jnp.int32), (BM, 16))
    acc = jnp.where(mask[:, 0:WORDS_PER_ROW], jnp.int32(1), jnp.int32(0))
    for s in range(1, 32):
        bit = jnp.int32(np.uint32(1 << s).view(np.int32))
        acc = acc + jnp.where(
            mask[:, s * WORDS_PER_ROW:(s + 1) * WORDS_PER_ROW],
            bit, jnp.int32(0))
    packed_ref[...] = acc


def _offsets_kernel(cnt_ref, out_ref):
    cnt = cnt_ref[...].astype(jnp.float32)
    k = lax.broadcasted_iota(jnp.int32, (128, 128), 0)
    l = lax.broadcasted_iota(jnp.int32, (128, 128), 1)
    tri = (k <= l).astype(jnp.float32)
    incl = jnp.dot(cnt, tri, precision=lax.Precision.HIGHEST)
    i = lax.broadcasted_iota(jnp.int32, (16, 16), 0)
    j = lax.broadcasted_iota(jnp.int32, (16, 16), 1)
    strict = (j < i).astype(jnp.float32)
    row_tot = incl[:, 127:128]
    row_off = jnp.dot(strict, row_tot, precision=lax.Precision.HIGHEST)
    excl = row_off + incl - cnt
    out_ref[0:16, :] = excl.astype(jnp.int32)
    total = row_off[15:16, 0:1] + incl[15:16, 127:128]
    out_ref[16:17, :] = jnp.broadcast_to(total, (1, 128)).astype(jnp.int32)


def _sc_body(packed_hbm, cnt_hbm, out_hbm, mbits, cvm,
             rowbuf_a, rowbuf_b, ssem_a, ssem_b):
    info = plsc.get_sparse_core_info()
    nc = info.num_cores
    wid = lax.axis_index("s") * nc + lax.axis_index("c")
    base_row = wid * ROWS_PER_W
    lanes = lax.iota(jnp.int32, 16)
    ones16 = jnp.ones((16,), jnp.int32)

    pltpu.sync_copy(packed_hbm.at[pl.ds(base_row, ROWS_PER_W), :], mbits)
    pltpu.sync_copy(cnt_hbm.at[pl.ds(base_row, ROWS_PER_W), :], cvm)

    def step(r, row, rowbuf, ssem):
        @pl.when(r >= 2)
        def _():
            pltpu.make_async_copy(rowbuf, out_hbm.at[row - 2], ssem).wait()

        cntv = cvm[r, pl.ds(0, 16)]

        @plsc.parallel_loop(0, N_REF // 16, 1, unroll=16,
                            carry=(lanes, -ones16, cntv + lanes))
        def body(c, carry_in):
            jv, trunm1, cj = carry_in
            g = c & (GROUPS - 1)
            s = c >> 4
            w = mbits[r, pl.ds(g * 16, 16)]
            m01 = (w >> s) & 1
            mb = m01 > 0
            incl = plsc.cumsum(m01)
            pos = jnp.where(mb, trunm1 + incl, cj - incl)
            plsc.store_scatter(rowbuf, [pos], jv)
            pc = plsc.all_reduce_population_count(mb)
            return jv + 16, trunm1 + pc, cj + (16 - pc)

        pltpu.async_copy(rowbuf, out_hbm.at[row], ssem)

    def do_row(r, carry):
        row = base_row + r

        @pl.when((r & 1) == 0)
        def _():
            step(r, row, rowbuf_a, ssem_a)

        @pl.when((r & 1) == 1)
        def _():
            step(r, row, rowbuf_b, ssem_b)

        return carry

    lax.fori_loop(0, ROWS_PER_W, do_row, 0)

    pltpu.make_async_copy(
        rowbuf_a, out_hbm.at[base_row + ROWS_PER_W - 2], ssem_a).wait()
    pltpu.make_async_copy(
        rowbuf_b, out_hbm.at[base_row + ROWS_PER_W - 1], ssem_b).wait()


@functools.partial(jax.jit, static_argnums=())
def kernel(ref_positions, query_positions):
    rsqt = jnp.sum(ref_positions * ref_positions, axis=1, keepdims=True).T

    grid = N_QUERY // BM
    mask, packed, counts_spl = pl.pallas_call(
        _mask_kernel,
        grid=(grid,),
        in_specs=[
            pl.BlockSpec((BM, 3), lambda i: (i, 0)),
            pl.BlockSpec((N_REF, 3), lambda i: (0, 0)),
            pl.BlockSpec((1, N_REF), lambda i: (0, 0)),
        ],
        out_specs=[
            pl.BlockSpec((BM, N_REF), lambda i: (i, 0)),
            pl.BlockSpec((BM, WORDS_PER_ROW), lambda i: (i, 0)),
            pl.BlockSpec((BM, 16), lambda i: (i, 0)),
        ],
        out_shape=[
            jax.ShapeDtypeStruct((N_QUERY, N_REF), jnp.bool_),
            jax.ShapeDtypeStruct((N_QUERY, WORDS_PER_ROW), jnp.int32),
            jax.ShapeDtypeStruct((N_QUERY, 16), jnp.int32),
        ],
    )(query_positions, ref_positions, rsqt)

    counts = counts_spl[:, 0]

    offsets_2d = pl.pallas_call(
        _offsets_kernel,
        out_shape=jax.ShapeDtypeStruct((17, 128), jnp.int32),
    )(counts.reshape(16, 128))
    offsets = offsets_2d.reshape(-1)[: N_QUERY + 1]

    mesh = plsc.VectorSubcoreMesh(core_axis_name="c", subcore_axis_name="s")
    neighbor_index = pl.kernel(
        _sc_body,
        out_type=jax.ShapeDtypeStruct((N_QUERY, N_REF), jnp.int32),
        mesh=mesh,
        compiler_params=pltpu.CompilerParams(needs_layout_passes=False),
        scratch_types=[
            pltpu.VMEM((ROWS_PER_W, WORDS_PER_ROW), jnp.int32),
            pltpu.VMEM((ROWS_PER_W, 16), jnp.int32),
            pltpu.VMEM((N_REF,), jnp.int32),
            pltpu.VMEM((N_REF,), jnp.int32),
            pltpu.SemaphoreType.DMA,
            pltpu.SemaphoreType.DMA,
        ],
    )(packed, counts_spl)

    return neighbor_index, counts, offsets, mask
